# Initial kernel scaffold; baseline (speedup 1.0000x reference)
import jax
import jax.numpy as jnp
from jax.experimental import pallas as pl


def kernel(x, W1, b1, W2, b2, w_metric):
    raise NotImplementedError("write your pallas kernel here")


# speedup vs baseline: 5.7408x; 5.7408x over previous
import functools

import jax
import jax.numpy as jnp
from jax.experimental import pallas as pl
from jax.experimental.pallas import tpu as pltpu

_D = 4096
_N = 64
_LO = (_D - _N) // 2
_BLK = 512
_NBLK = _D // _BLK


def _mlp_argmax_kernel(xw_ref, w1w_ref, b1_ref, b2_ref, wm_ref, w2_ref,
                       out_ref, h_ref, t2_ref, m_ref):
    j = pl.program_id(0)

    @pl.when(j == 0)
    def _init():
        pre = jnp.dot(xw_ref[...], w1w_ref[...],
                      preferred_element_type=jnp.float32)
        h_ref[...] = jnp.maximum(pre + b1_ref[...] + w1w_ref[...], 0.0)
        m_ref[...] = jnp.zeros_like(m_ref)

    t2_blk = jnp.dot(h_ref[...], w2_ref[...],
                     preferred_element_type=jnp.float32) + b2_ref[...]
    t2_ref[:, pl.ds(j * _BLK, _BLK)] = t2_blk
    m_ref[...] += jnp.sum(t2_blk * wm_ref[...], axis=1, keepdims=True)

    @pl.when(j == _NBLK - 1)
    def _fin():
        metric = m_ref[...]
        mmax = jnp.max(metric)
        iota = jax.lax.broadcasted_iota(jnp.int32, (_N, 1), 0)
        idx = jnp.min(jnp.where(metric == mmax, iota, _N))
        onehot = (iota == idx).astype(jnp.float32)
        out_ref[...] = jnp.sum(t2_ref[...] * onehot, axis=0, keepdims=True)


@jax.jit
def kernel(x, W1, b1, W2, b2, w_metric):
    xw = jax.lax.slice(x, (_LO,), (_LO + _N,)).reshape(1, _N)
    w1w = jax.lax.slice(W1, (_LO, 0), (_LO + _N, _D))
    b1r = b1.reshape(1, _D)
    b2r = b2.reshape(1, _D)
    wmr = w_metric.reshape(1, _D)

    out = pl.pallas_call(
        _mlp_argmax_kernel,
        grid=(_NBLK,),
        in_specs=[
            pl.BlockSpec((1, _N), lambda j: (0, 0)),
            pl.BlockSpec((_N, _D), lambda j: (0, 0)),
            pl.BlockSpec((1, _D), lambda j: (0, 0)),
            pl.BlockSpec((1, _BLK), lambda j: (0, j)),
            pl.BlockSpec((1, _BLK), lambda j: (0, j)),
            pl.BlockSpec((_D, _BLK), lambda j: (0, j)),
        ],
        out_specs=pl.BlockSpec((1, _D), lambda j: (0, 0)),
        out_shape=jax.ShapeDtypeStruct((1, _D), jnp.float32),
        scratch_shapes=[
            pltpu.VMEM((_N, _D), jnp.float32),
            pltpu.VMEM((_N, _D), jnp.float32),
            pltpu.VMEM((_N, 1), jnp.float32),
        ],
        compiler_params=pltpu.CompilerParams(
            dimension_semantics=("arbitrary",),
        ),
    )(xw, w1w, b1r, b2r, wmr, W2)
    return out.reshape(_D)
